# Initial kernel scaffold; baseline (speedup 1.0000x reference)
import jax
import jax.numpy as jnp
from jax.experimental import pallas as pl


def kernel(X, sparse_rows, sparse_cols, sparse_vals, X0, ui_adj, W1, b1, W2, b2, W3, b3, g0, be0, g1, be1):
    raise NotImplementedError("write your pallas kernel here")


# speedup vs baseline: 3.3923x; 3.3923x over previous
import functools

import jax
import jax.numpy as jnp
from jax import lax
from jax.experimental import pallas as pl
from jax.experimental.pallas import tpu as pltpu
from jax.experimental.pallas import tpu_sc as plsc

_D = 128
_CH = 128
_NTILES = 32
_SLOPE = 0.5
_ALPHA = 0.5


def _make_sc_pass(T, nnzp):
    per_tile = nnzp // _NTILES
    n_chunks = per_tile // _CH
    rpt = T // 16
    mesh = plsc.VectorSubcoreMesh(core_axis_name="c", subcore_axis_name="s")

    @functools.partial(
        pl.kernel,
        out_type=jax.ShapeDtypeStruct((2, T, _D), jnp.float32),
        mesh=mesh,
        scratch_types=[
            pltpu.VMEM((_CH,), jnp.int32),
            pltpu.VMEM((_CH,), jnp.int32),
            pltpu.VMEM((_CH,), jnp.float32),
            pltpu.VMEM((_CH, _D), jnp.float32),
            pltpu.VMEM_SHARED((T, _D), jnp.float32),
            pltpu.SemaphoreType.DMA,
        ],
    )
    def sc_pass(src, gidx, sidx, vals, zeros, out,
                gidx_v, sidx_v, vals_v, rows_v, acc, sem):
        cid = lax.axis_index("c")
        sid = lax.axis_index("s")
        wid = cid * 16 + sid

        pltpu.sync_copy(zeros.at[pl.ds(sid * rpt, rpt)],
                        acc.at[pl.ds(sid * rpt, rpt)])
        plsc.subcore_barrier()

        base = wid * per_tile

        def chunk(i, carry):
            off = base + i * _CH
            pltpu.sync_copy(gidx.at[pl.ds(off, _CH)], gidx_v)
            pltpu.sync_copy(sidx.at[pl.ds(off, _CH)], sidx_v)
            pltpu.sync_copy(vals.at[pl.ds(off, _CH)], vals_v)
            pltpu.async_copy(src.at[gidx_v], rows_v, sem).wait()

            def grp(g, c2):
                vv = vals_v[pl.ds(g * 16, 16)]
                for rr in range(16):
                    v = vv[rr]
                    r = g * 16 + rr
                    for j in range(8):
                        sl = pl.ds(j * 16, 16)
                        rows_v[r, sl] = rows_v[r, sl] * v
                return c2

            lax.fori_loop(0, _CH // 16, grp, 0)
            pltpu.sync_copy(rows_v, acc.at[sidx_v], add=True)
            return carry

        lax.fori_loop(0, n_chunks, chunk, 0)
        plsc.subcore_barrier()
        pltpu.sync_copy(acc.at[pl.ds(sid * rpt, rpt)],
                        out.at[cid, pl.ds(sid * rpt, rpt)])

    return sc_pass


def _dot(a, b):
    return lax.dot_general(a, b, (((1,), (0,)), ((), ())),
                           precision=lax.Precision.HIGHEST,
                           preferred_element_type=jnp.float32)


def _ln(x, g, b):
    mu = jnp.mean(x, axis=-1, keepdims=True)
    var = jnp.mean((x - mu) ** 2, axis=-1, keepdims=True)
    return (x - mu) / jnp.sqrt(var + 1e-5) * g + b


def _leaky(x):
    return jnp.where(x >= 0, x, _SLOPE * x)


def _t1_body(x_ref, w_ref, b_ref, o_ref):
    o_ref[...] = _dot(x_ref[...], w_ref[...]) + b_ref[...]


def _comb_body(a_ref, b_ref, o_ref):
    o_ref[...] = a_ref[...] + b_ref[...]


def _t2_body(p0_ref, p1_ref, xve_ref, x_ref, w2a_ref, w2b_ref, b2_ref,
             g0_ref, be0_ref, o_ref):
    xv = _leaky(p0_ref[...] + p1_ref[...])
    xe = _ln(xv, g0_ref[...], be0_ref[...]) + xve_ref[...]
    o_ref[...] = _dot(x_ref[...], w2a_ref[...]) + _dot(xe, w2b_ref[...]) \
        + b2_ref[...]


def _t3_body(p0_ref, p1_ref, xev_ref, x0_ref, w3_ref, b3_ref,
             g1_ref, be1_ref, o_ref):
    xv = _leaky(p0_ref[...] + p1_ref[...])
    x_v = _ln(xv, g1_ref[...], be1_ref[...]) + xev_ref[...]
    xmix = (1.0 - _ALPHA) * x_v + _ALPHA * x0_ref[...]
    o_ref[...] = _dot(xmix, w3_ref[...]) + b3_ref[...]


def _row_block_call(body, n_rows, blk, row_args, full_args, out_cols=_D):
    grid = (n_rows // blk,)
    in_specs = (
        [pl.BlockSpec((blk, a.shape[1]), lambda i: (i, 0)) for a in row_args]
        + [pl.BlockSpec(a.shape, lambda i: (0, 0)) for a in full_args]
    )
    return pl.pallas_call(
        body,
        grid=grid,
        in_specs=in_specs,
        out_specs=pl.BlockSpec((blk, out_cols), lambda i: (i, 0)),
        out_shape=jax.ShapeDtypeStruct((n_rows, out_cols), jnp.float32),
    )(*row_args, *full_args)


def kernel(X, sparse_rows, sparse_cols, sparse_vals, X0, ui_adj,
           W1, b1, W2, b2, W3, b3, g0, be0, g1, be1):
    n, d = X.shape
    m = 5000
    mp = ((m + 127) // 128) * 128
    np_ = ((n + 127) // 128) * 128
    nnz = sparse_rows.shape[0]
    step = _NTILES * _CH
    nnzp = ((nnz + step - 1) // step) * step

    pad = nnzp - nnz
    rows_p = jnp.concatenate([sparse_rows, jnp.zeros((pad,), jnp.int32)])
    cols_p = jnp.concatenate([sparse_cols, jnp.zeros((pad,), jnp.int32)])
    vals_p = jnp.concatenate([sparse_vals, jnp.zeros((pad,), jnp.float32)])

    zeros_m = jnp.zeros((mp, d), jnp.float32)
    zeros_n = jnp.zeros((np_, d), jnp.float32)

    b1r = b1.reshape(1, d)
    b2r = b2.reshape(1, d)
    b3r = b3.reshape(1, d)
    g0r = g0.reshape(1, d)
    be0r = be0.reshape(1, d)
    g1r = g1.reshape(1, d)
    be1r = be1.reshape(1, d)
    w2a = W2[:d]
    w2b = W2[d:]

    sc_to_edges = _make_sc_pass(mp, nnzp)
    sc_to_nodes = _make_sc_pass(np_, nnzp)

    xve = _row_block_call(_t1_body, n, 1000, [X], [W1, b1r])

    pa = sc_to_edges(xve, rows_p, cols_p, vals_p, zeros_m)
    xe_edges = _row_block_call(_comb_body, mp, mp, [pa[0], pa[1]], [])
    pb = sc_to_nodes(xe_edges, cols_p, rows_p, vals_p, zeros_n)

    xev = _row_block_call(_t2_body, n, 1000, [pb[0, :n], pb[1, :n], xve, X],
                          [w2a, w2b, b2r, g0r, be0r])

    pc = sc_to_edges(xev, rows_p, cols_p, vals_p, zeros_m)
    xe_edges2 = _row_block_call(_comb_body, mp, mp, [pc[0], pc[1]], [])
    pd = sc_to_nodes(xe_edges2, cols_p, rows_p, vals_p, zeros_n)

    out = _row_block_call(_t3_body, n, 1000, [pd[0, :n], pd[1, :n], xev, X0],
                          [W3, b3r, g1r, be1r])
    return out
